# Initial kernel scaffold; baseline (speedup 1.0000x reference)
import jax
import jax.numpy as jnp
from jax.experimental import pallas as pl


def kernel(x_upper, x_lower, W_lu, b_lu, g_lu, be_lu, W_ul, b_ul, g_ul, be_ul, alpha_p, beta_p):
    raise NotImplementedError("write your pallas kernel here")


# speedup vs baseline: 4.4569x; 4.4569x over previous
import functools

import jax
import jax.numpy as jnp
from jax.experimental import pallas as pl
from jax.experimental.pallas import tpu as pltpu

_N = 100000
_D = 128
_BLOCK = 2000


def _body(up_ref, lo_ref, wlut_ref, blu_ref, glu_ref, belu_ref,
          wult_ref, bul_ref, gul_ref, beul_ref, ap_ref, bp_ref,
          out_up_ref, out_lo_ref):
    up = up_ref[...]
    lo = lo_ref[...]

    def branch(x, wt_ref, b_ref, g_ref, be_ref):
        h = jnp.dot(x.astype(jnp.bfloat16), wt_ref[...],
                    preferred_element_type=jnp.float32) + b_ref[...]
        mu = jnp.mean(h, axis=-1, keepdims=True)
        xc = h - mu
        var = jnp.mean(xc * xc, axis=-1, keepdims=True)
        xn = xc * jax.lax.rsqrt(var + 1e-5)
        y = xn * g_ref[...] + be_ref[...]
        return 0.5 * y * (1.0 + jax.lax.erf(y * 0.7071067811865476))

    lower_t = branch(lo, wlut_ref, blu_ref, glu_ref, belu_ref)
    upper_t = branch(up, wult_ref, bul_ref, gul_ref, beul_ref)

    alpha = jax.nn.sigmoid(ap_ref[...])
    beta = jax.nn.sigmoid(bp_ref[...])
    out_up_ref[...] = alpha * up + (1.0 - alpha) * lower_t
    out_lo_ref[...] = beta * lo + (1.0 - beta) * upper_t


@functools.partial(jax.jit, static_argnames=())
def kernel(x_upper, x_lower, W_lu, b_lu, g_lu, be_lu,
           W_ul, b_ul, g_ul, be_ul, alpha_p, beta_p):
    n, d = x_upper.shape
    block = _BLOCK if n % _BLOCK == 0 else n
    grid = (n // block,)

    row_spec = pl.BlockSpec((block, d), lambda i: (i, 0))
    full_spec = pl.BlockSpec((d, d), lambda i: (0, 0))
    vec_spec = pl.BlockSpec((1, d), lambda i: (0, 0))

    wlut = W_lu.T.astype(jnp.bfloat16)
    wult = W_ul.T.astype(jnp.bfloat16)

    out_up, out_lo = pl.pallas_call(
        _body,
        grid=grid,
        in_specs=[row_spec, row_spec,
                  full_spec, vec_spec, vec_spec, vec_spec,
                  full_spec, vec_spec, vec_spec, vec_spec,
                  vec_spec, vec_spec],
        out_specs=[row_spec, row_spec],
        out_shape=[jax.ShapeDtypeStruct((n, d), jnp.float32),
                   jax.ShapeDtypeStruct((n, d), jnp.float32)],
        compiler_params=pltpu.CompilerParams(
            dimension_semantics=("arbitrary",),
        ),
    )(x_upper, x_lower,
      wlut, b_lu.reshape(1, d), g_lu.reshape(1, d), be_lu.reshape(1, d),
      wult, b_ul.reshape(1, d), g_ul.reshape(1, d), be_ul.reshape(1, d),
      alpha_p, beta_p)
    return (out_up, out_lo)
